# Initial kernel scaffold; baseline (speedup 1.0000x reference)
import jax
import jax.numpy as jnp
from jax.experimental import pallas as pl


def kernel(element, x, inducing_x, alpha, lengthscales):
    raise NotImplementedError("write your pallas kernel here")


# speedup vs baseline: 11.9071x; 11.9071x over previous
import functools

import jax
import jax.numpy as jnp
from jax.experimental import pallas as pl

E = 8
N = 4096
P = 128
D = 64
BN = 512


def _block_kernel(elem_ref, x_ref, u_ref, a_ref, ls_ref, out_ref):
    xv = x_ref[...]
    elem = elem_ref[0, 0, :]
    acc = jnp.zeros((BN,), dtype=jnp.float32)
    for e in range(E):
        w = jnp.exp(-ls_ref[e, :])
        u = u_ref[e]
        xw = xv * w[None, :]
        xsq = jnp.sum(xw * xv, axis=1)
        usq = jnp.sum(u * u * w[None, :], axis=1)
        cross = jnp.dot(xw, u.T, preferred_element_type=jnp.float32)
        diff = xsq[:, None] + usq[None, :] - 2.0 * cross
        esd = jnp.exp(-diff)
        energies = jnp.dot(esd, a_ref[e, :],
                           preferred_element_type=jnp.float32)
        acc = jnp.where(elem == e, energies, acc)
    out_ref[...] = acc


@jax.jit
def kernel(element, x, inducing_x, alpha, lengthscales):
    n = x.shape[0]
    nb = n // BN
    elem3 = element.astype(jnp.int32).reshape(nb, 1, BN)
    grid = (nb,)
    out = pl.pallas_call(
        _block_kernel,
        grid=grid,
        in_specs=[
            pl.BlockSpec((1, 1, BN), lambda i: (i, 0, 0)),
            pl.BlockSpec((BN, D), lambda i: (i, 0)),
            pl.BlockSpec((E, P, D), lambda i: (0, 0, 0)),
            pl.BlockSpec((E, P), lambda i: (0, 0)),
            pl.BlockSpec((E, D), lambda i: (0, 0)),
        ],
        out_specs=pl.BlockSpec((BN,), lambda i: (i,)),
        out_shape=jax.ShapeDtypeStruct((n,), jnp.float32),
    )(elem3, x, inducing_x, alpha, lengthscales)
    return out


# speedup vs baseline: 19.9628x; 1.6766x over previous
import jax
import jax.numpy as jnp
from jax.experimental import pallas as pl
from jax.experimental.pallas import tpu as pltpu

E = 8
N = 4096
P = 128
D = 64
BN = 512


def _block_kernel(elem_ref, x_ref, u_ref, abd_ref, ls_ref, out_ref,
                  uw_ref, usq_ref):
    @pl.when(pl.program_id(0) == 0)
    def _prep():
        w = jnp.exp(-ls_ref[...])
        for e in range(E):
            uw = u_ref[e] * w[e][None, :]
            uw_ref[e] = uw
            usq_ref[e] = jnp.sum(u_ref[e] * uw, axis=1, keepdims=True).T

    xv = x_ref[...]
    xx = xv * xv
    w = jnp.exp(-ls_ref[...])
    esd = []
    for e in range(E):
        xsq = jnp.sum(xx * w[e][None, :], axis=1, keepdims=True)
        cross = jnp.dot(xv, uw_ref[e].T,
                        preferred_element_type=jnp.float32)
        diff = xsq + usq_ref[e] - 2.0 * cross
        esd.append(jnp.exp(-diff))
    esd_all = jnp.concatenate(esd, axis=1)
    h = jnp.dot(esd_all, abd_ref[...],
                preferred_element_type=jnp.float32)
    elem = elem_ref[0, 0, :]
    onehot = (elem[:, None] ==
              jax.lax.broadcasted_iota(jnp.int32, (BN, E), 1))
    out_ref[...] = jnp.sum(jnp.where(onehot, h, 0.0), axis=1)


@jax.jit
def kernel(element, x, inducing_x, alpha, lengthscales):
    n = x.shape[0]
    nb = n // BN
    elem3 = element.astype(jnp.int32).reshape(nb, 1, BN)
    a_flat = alpha.reshape(E * P).astype(jnp.float32)
    blk = (jnp.arange(E * P)[:, None] // P) == jnp.arange(E)[None, :]
    a_bd = jnp.where(blk, a_flat[:, None], 0.0)
    out = pl.pallas_call(
        _block_kernel,
        grid=(nb,),
        in_specs=[
            pl.BlockSpec((1, 1, BN), lambda i: (i, 0, 0)),
            pl.BlockSpec((BN, D), lambda i: (i, 0)),
            pl.BlockSpec((E, P, D), lambda i: (0, 0, 0)),
            pl.BlockSpec((E * P, E), lambda i: (0, 0)),
            pl.BlockSpec((E, D), lambda i: (0, 0)),
        ],
        out_specs=pl.BlockSpec((BN,), lambda i: (i,)),
        out_shape=jax.ShapeDtypeStruct((n,), jnp.float32),
        scratch_shapes=[
            pltpu.VMEM((E, P, D), jnp.float32),
            pltpu.VMEM((E, 1, P), jnp.float32),
        ],
    )(elem3, x, inducing_x, a_bd, lengthscales)
    return out
